# baseline (speedup 1.0000x reference)
import functools

import jax
import jax.numpy as jnp
from jax import lax
from jax.experimental import pallas as pl
from jax.experimental.pallas import tpu as pltpu
from jax.experimental.pallas import tpu_sc as plsc

D = 64
SCALE = 8.0
NC, NS, L = 2, 16, 16
NW = NC * NS
B = 4096 * 200
BPW = B // NW
CHUNK = 128
NCHUNK = BPW // CHUNK


def _emb_body(tokens_hbm, table_hbm, out_hbm,
              idx_v, in0, in1, out0, out1, g0, g1, s0, s1):
  wid = lax.axis_index("s") * NC + lax.axis_index("c")
  base = wid * BPW

  pltpu.sync_copy(tokens_hbm.at[wid], idx_v)

  ins = (in0, in1)
  outs = (out0, out1)
  gsems = (g0, g1)
  ssems = (s0, s1)

  def start_gather(g, b):
    pltpu.make_async_copy(table_hbm.at[idx_v.at[g]], ins[b], gsems[b]).start()

  def wait_gather(g, b):
    pltpu.make_async_copy(table_hbm.at[idx_v.at[g]], ins[b], gsems[b]).wait()

  def start_store(g, b):
    pltpu.make_async_copy(
        outs[b], out_hbm.at[pl.ds(base + g * CHUNK, CHUNK)], ssems[b]).start()

  def wait_store(b):
    pltpu.make_async_copy(
        outs[b], out_hbm.at[pl.ds(base, CHUNK)], ssems[b]).wait()

  def scale(b):
    ib = ins[b]
    ob = outs[b]

    def row(r, carry):
      for c in range(D // L):
        ob[r, pl.ds(c * L, L)] = ib[r, pl.ds(c * L, L)] * SCALE
      return carry

    lax.fori_loop(0, CHUNK, row, 0)

  start_gather(0, 0)
  start_gather(1, 1)

  for g in (0, 1):
    b = g
    wait_gather(g, b)
    scale(b)
    start_gather(g + 2, b)
    start_store(g, b)

  def mid(k, carry):
    for b in range(2):
      g = 2 * k + b
      wait_gather(g, b)
      wait_store(b)
      scale(b)
      start_gather(g + 2, b)
      start_store(g, b)
    return carry

  lax.fori_loop(1, NCHUNK // 2 - 1, mid, 0)

  for g in (NCHUNK - 2, NCHUNK - 1):
    b = g % 2
    wait_gather(g, b)
    wait_store(b)
    scale(b)
    start_store(g, b)

  for b in range(2):
    wait_store(b)


_emb_call = functools.partial(
    pl.kernel,
    out_type=jax.ShapeDtypeStruct((B, D), jnp.float32),
    mesh=plsc.VectorSubcoreMesh(core_axis_name="c", subcore_axis_name="s"),
    compiler_params=pltpu.CompilerParams(use_tc_tiling_on_sc=False),
    scratch_types=[
        pltpu.VMEM((NCHUNK, CHUNK), jnp.int32),
        pltpu.VMEM((CHUNK, D), jnp.float32),
        pltpu.VMEM((CHUNK, D), jnp.float32),
        pltpu.VMEM((CHUNK, D), jnp.float32),
        pltpu.VMEM((CHUNK, D), jnp.float32),
        pltpu.SemaphoreType.DMA,
        pltpu.SemaphoreType.DMA,
        pltpu.SemaphoreType.DMA,
        pltpu.SemaphoreType.DMA,
    ],
)(_emb_body)


@jax.jit
def kernel(tokens, table):
  bsz, seq = tokens.shape
  toks = tokens.astype(jnp.int32).reshape(NW, NCHUNK, CHUNK)
  out = _emb_call(toks, table)
  return out.reshape(bsz, seq, D)


# speedup vs baseline: 1.0186x; 1.0186x over previous
import functools

import jax
import jax.numpy as jnp
from jax import lax
from jax.experimental import pallas as pl
from jax.experimental.pallas import tpu as pltpu
from jax.experimental.pallas import tpu_sc as plsc

D = 64
SCALE = 8.0
NC, NS, L = 2, 16, 16
NW = NC * NS
BATCH = 4096
SEQ = 200
BPERW = BATCH // NW
CB = 2
NCHUNK = BPERW // CB
IDXW = 100


def _emb_body(tokens_hbm, table_hbm, out_hbm,
              idx_v, in0, in1, out0, out1, g0, g1, s0, s1):
  wid = lax.axis_index("s") * NC + lax.axis_index("c")
  bbase = wid * BPERW

  pltpu.sync_copy(tokens_hbm.at[wid], idx_v)

  ins = (in0, in1)
  outs = (out0, out1)
  gsems = (g0, g1)
  ssems = (s0, s1)

  def start_gather(j, b):
    for i2 in range(CB):
      for h in range(2):
        pltpu.make_async_copy(
            table_hbm.at[idx_v.at[j, 2 * i2 + h]],
            ins[b].at[i2, pl.ds(h * IDXW, IDXW)],
            gsems[b]).start()

  def wait_gather(j, b):
    for i2 in range(CB):
      for h in range(2):
        pltpu.make_async_copy(
            table_hbm.at[idx_v.at[j, 2 * i2 + h]],
            ins[b].at[i2, pl.ds(h * IDXW, IDXW)],
            gsems[b]).wait()

  def start_store(j, b):
    pltpu.make_async_copy(
        outs[b], out_hbm.at[pl.ds(bbase + j * CB, CB)], ssems[b]).start()

  def wait_store(b):
    pltpu.make_async_copy(
        outs[b], out_hbm.at[pl.ds(bbase, CB)], ssems[b]).wait()

  def scale(b):
    ib = ins[b]
    ob = outs[b]

    def row(r, carry):
      for i2 in range(CB):
        for c in range(D // L):
          sl = pl.ds(c * L, L)
          ob[i2, r, sl] = ib[i2, r, sl] * SCALE
      return carry

    lax.fori_loop(0, SEQ, row, 0)

  start_gather(0, 0)
  start_gather(1, 1)

  for j in (0, 1):
    b = j
    wait_gather(j, b)
    scale(b)
    start_gather(j + 2, b)
    start_store(j, b)

  def mid(k, carry):
    for b in range(2):
      j = 2 * k + b
      wait_gather(j, b)
      wait_store(b)
      scale(b)
      start_gather(j + 2, b)
      start_store(j, b)
    return carry

  lax.fori_loop(1, NCHUNK // 2 - 1, mid, 0)

  for j in (NCHUNK - 2, NCHUNK - 1):
    b = j % 2
    wait_gather(j, b)
    wait_store(b)
    scale(b)
    start_store(j, b)

  for b in range(2):
    wait_store(b)


_emb_call = functools.partial(
    pl.kernel,
    out_type=jax.ShapeDtypeStruct((BATCH, SEQ, D), jnp.float32),
    mesh=plsc.VectorSubcoreMesh(core_axis_name="c", subcore_axis_name="s"),
    compiler_params=pltpu.CompilerParams(use_tc_tiling_on_sc=False),
    scratch_types=[
        pltpu.VMEM((NCHUNK, 2 * CB, IDXW), jnp.int32),
        pltpu.VMEM((CB, SEQ, D), jnp.float32),
        pltpu.VMEM((CB, SEQ, D), jnp.float32),
        pltpu.VMEM((CB, SEQ, D), jnp.float32),
        pltpu.VMEM((CB, SEQ, D), jnp.float32),
        pltpu.SemaphoreType.DMA,
        pltpu.SemaphoreType.DMA,
        pltpu.SemaphoreType.DMA,
        pltpu.SemaphoreType.DMA,
    ],
)(_emb_body)


@jax.jit
def kernel(tokens, table):
  toks = tokens.astype(jnp.int32).reshape(NW, NCHUNK, 2 * CB, IDXW)
  return _emb_call(toks, table)
